# Initial kernel scaffold; baseline (speedup 1.0000x reference)
import jax
import jax.numpy as jnp
from jax.experimental import pallas as pl


def kernel(ip1_idx, ip1_table, ip2_idx, ip2_table, ip3_idx, ip3_table, url_idx, url_table, aurl_idx, aurl_table, regionid_idx, regionid_table, cityid_idx, cityid_table, adexchange_idx, adexchange_table, adslotw_idx, adslotw_table, adsloth_idx, adsloth_table, adslotv_idx, adslotv_table, adslotfp_idx, adslotfp_table, creativeid_idx, creativeid_table, bidprice_idx, bidprice_table, payprice_idx, payprice_table, userids_idx, userids_table, W, b):
    raise NotImplementedError("write your pallas kernel here")


# speedup vs baseline: 7.4207x; 7.4207x over previous
import functools

import jax
import jax.numpy as jnp
from jax import lax
from jax.experimental import pallas as pl
from jax.experimental.pallas import tpu as pltpu
from jax.experimental.pallas import tpu_sc as plsc

_B = 4096
_HIST = 20
_NC, _NS, _L = 2, 16, 16
_NW = _NC * _NS
_BPW = _B // _NW
_RP = 2048
_KP = 96
_NCLS = 8

_VOCABS = [256, 256, 256, 2, 2, 35, 370, 9, 21, 14, 7, 275, 57, 2, 295]
_DIMS = [8, 8, 8, 1, 1, 6, 9, 4, 5, 4, 3, 9, 6, 1, 9]
_UVOCAB, _UDIM = 69, 7
_NF = len(_VOCABS)
_NJ = _NF + _HIST

_ROW_OFF = [0] * _NF
for _i in range(1, _NF):
    _ROW_OFF[_i] = _ROW_OFF[_i - 1] + _VOCABS[_i - 1]
_UROW = _ROW_OFF[-1] + _VOCABS[-1]
_COL_OFF = [0] * _NF
for _i in range(1, _NF):
    _COL_OFF[_i] = _COL_OFF[_i - 1] + _DIMS[_i - 1]
_UCOL = _COL_OFF[-1] + _DIMS[-1]
_BIAS_COL = _UCOL + _UDIM


def _fuse_tables_body(w_ref, p_ref, t_ref):
    t = lax.dot_general(
        w_ref[...], p_ref[...],
        dimension_numbers=(((0,), (1,)), ((), ())),
        preferred_element_type=jnp.float32,
    )
    r = lax.broadcasted_iota(jnp.int32, (_NCLS, _RP), 1)
    t_ref[...] = jnp.where(r >= _UROW, t * (1.0 / _HIST), t)


_fuse_tables = pl.pallas_call(
    _fuse_tables_body,
    out_shape=jax.ShapeDtypeStruct((_NCLS, _RP), jnp.float32),
)


@functools.cache
def _make_sc_bag():
    return pl.kernel(
        _sc_bag_body,
        mesh=plsc.VectorSubcoreMesh(core_axis_name="c", subcore_axis_name="s"),
        out_type=jax.ShapeDtypeStruct((2 * _NW, _BPW), jnp.float32),
        scratch_types=[
            pltpu.VMEM((_NJ, _BPW), jnp.int32),
            pltpu.VMEM((_RP,), jnp.float32),
            pltpu.VMEM((_RP,), jnp.float32),
            pltpu.VMEM((_BPW,), jnp.float32),
            pltpu.VMEM((_BPW,), jnp.float32),
        ],
        compiler_params=pltpu.CompilerParams(needs_layout_passes=False),
    )


def _sc_bag_body(idx_hbm, t_hbm, out_hbm, idx_v, t0_v, t1_v, o0_v, o1_v):
    w = lax.axis_index("s") * _NC + lax.axis_index("c")
    pltpu.sync_copy(idx_hbm.at[w], idx_v)
    pltpu.sync_copy(t_hbm.at[0], t0_v)
    pltpu.sync_copy(t_hbm.at[1], t1_v)
    offs = _ROW_OFF + [_UROW] * _HIST
    for g in range(_BPW // _L):
        sl = pl.ds(g * _L, _L)
        a0 = jnp.zeros((_L,), jnp.float32)
        a1 = jnp.zeros((_L,), jnp.float32)
        for j in range(_NJ):
            iv = idx_v[j, sl] + offs[j]
            a0 = a0 + plsc.load_gather(t0_v, [iv])
            a1 = a1 + plsc.load_gather(t1_v, [iv])
        m = jnp.maximum(a0, a1)
        e = jnp.exp(-jnp.abs(a0 - a1))
        z = e / (e + 2.0)
        z2 = z * z
        lse = m + 2.0 * z * (1.0 + z2 * (
            (1.0 / 3.0) + z2 * (0.2 + z2 * ((1.0 / 7.0) + z2 * (1.0 / 9.0)))))
        o0_v[sl] = a0 - lse
        o1_v[sl] = a1 - lse
    pltpu.sync_copy(o0_v, out_hbm.at[w])
    pltpu.sync_copy(o1_v, out_hbm.at[_NW + w])


def kernel(ip1_idx, ip1_table, ip2_idx, ip2_table, ip3_idx, ip3_table,
           url_idx, url_table, aurl_idx, aurl_table,
           regionid_idx, regionid_table, cityid_idx, cityid_table,
           adexchange_idx, adexchange_table, adslotw_idx, adslotw_table,
           adsloth_idx, adsloth_table, adslotv_idx, adslotv_table,
           adslotfp_idx, adslotfp_table, creativeid_idx, creativeid_table,
           bidprice_idx, bidprice_table, payprice_idx, payprice_table,
           userids_idx, userids_table, W, b):
    tables = [ip1_table, ip2_table, ip3_table, url_table, aurl_table,
              regionid_table, cityid_table, adexchange_table, adslotw_table,
              adsloth_table, adslotv_table, adslotfp_table, creativeid_table,
              bidprice_table, payprice_table]
    idxs = [ip1_idx, ip2_idx, ip3_idx, url_idx, aurl_idx, regionid_idx,
            cityid_idx, adexchange_idx, adslotw_idx, adsloth_idx, adslotv_idx,
            adslotfp_idx, creativeid_idx, bidprice_idx, payprice_idx]

    p = jnp.zeros((_RP, _KP), jnp.float32)
    for t, r0, c0, v, d in zip(tables, _ROW_OFF, _COL_OFF, _VOCABS, _DIMS):
        p = p.at[r0:r0 + v, c0:c0 + d].set(t)
    p = p.at[_UROW:_UROW + _UVOCAB, _UCOL:_UCOL + _UDIM].set(userids_table)
    p = p.at[0:_VOCABS[0], _BIAS_COL].set(1.0)

    wp = jnp.zeros((_KP, _NCLS), jnp.float32)
    wp = wp.at[0:_BIAS_COL, 0:2].set(W)
    wp = wp.at[_BIAS_COL, 0:2].set(b)

    t_full = _fuse_tables(wp, p)
    t2 = t_full[0:2, :]

    idx35 = jnp.concatenate(
        [jnp.stack(idxs, 0).astype(jnp.int32),
         userids_idx.T.astype(jnp.int32)], axis=0)
    idx_t = idx35.reshape(_NJ, _NW, _BPW).transpose(1, 0, 2)

    out = _make_sc_bag()(idx_t, t2)
    return out.reshape(2, _B).T


# speedup vs baseline: 11.0746x; 1.4924x over previous
import functools

import jax
import jax.numpy as jnp
from jax import lax
from jax.experimental import pallas as pl
from jax.experimental.pallas import tpu as pltpu
from jax.experimental.pallas import tpu_sc as plsc

_B = 4096
_HIST = 20
_NC, _NS, _L = 2, 16, 16
_NW = _NC * _NS
_BPW = _B // _NW
_NCLS = 8

_VOCABS = [256, 256, 256, 2, 2, 35, 370, 9, 21, 14, 7, 275, 57, 2, 295]
_DIMS = [8, 8, 8, 1, 1, 6, 9, 4, 5, 4, 3, 9, 6, 1, 9]
_UVOCAB, _UDIM = 69, 7
_NF = len(_VOCABS)

_ROW128 = []
_r = 0
for _v in _VOCABS:
    _ROW128.append(_r)
    _r += -(-_v // 128) * 128
_UROW128 = _r
_RP2 = _UROW128 + -(-_UVOCAB // 128) * 128

_COL_OFF = [0] * _NF
for _i in range(1, _NF):
    _COL_OFF[_i] = _COL_OFF[_i - 1] + _DIMS[_i - 1]
_UCOL = _COL_OFF[-1] + _DIMS[-1]


def _fuse_tables_body(*refs):
    w_ref, b_ref = refs[0], refs[1]
    tabs = refs[2:2 + _NF]
    ut_ref = refs[2 + _NF]
    t_ref = refs[3 + _NF]
    t_ref[...] = jnp.zeros((_NCLS, _RP2), jnp.float32)
    for i in range(_NF):
        blk = lax.dot_general(
            w_ref[_COL_OFF[i]:_COL_OFF[i] + _DIMS[i], :], tabs[i][...],
            dimension_numbers=(((0,), (1,)), ((), ())),
            preferred_element_type=jnp.float32)
        if i == 0:
            bias = lax.dot_general(
                b_ref[...], jnp.ones((1, _VOCABS[0]), jnp.float32),
                dimension_numbers=(((0,), (0,)), ((), ())),
                preferred_element_type=jnp.float32)
            blk = blk + bias
        t_ref[0:2, _ROW128[i]:_ROW128[i] + _VOCABS[i]] = blk
    ublk = lax.dot_general(
        w_ref[_UCOL:_UCOL + _UDIM, :], ut_ref[...],
        dimension_numbers=(((0,), (1,)), ((), ())),
        preferred_element_type=jnp.float32) * (1.0 / _HIST)
    t_ref[0:2, _UROW128:_UROW128 + _UVOCAB] = ublk


_fuse_tables = pl.pallas_call(
    _fuse_tables_body,
    out_shape=jax.ShapeDtypeStruct((_NCLS, _RP2), jnp.float32),
)


def _sc_bag_body(*refs):
    idx_hbm = refs[0:_NF]
    u_hbm, t_hbm, out_hbm = refs[_NF], refs[_NF + 1], refs[_NF + 2]
    idx_v, u_v, t0_v, t1_v, o_v, sem = refs[_NF + 3:]
    w = lax.axis_index("s") * _NC + lax.axis_index("c")
    base = w * _BPW
    copies = [pltpu.async_copy(ih.at[pl.ds(base, _BPW)], idx_v.at[f], sem)
              for f, ih in enumerate(idx_hbm)]
    copies.append(pltpu.async_copy(
        u_hbm.at[pl.ds(w * (_BPW * _HIST), _BPW * _HIST)], u_v, sem))
    copies.append(pltpu.async_copy(t_hbm.at[0], t0_v, sem))
    copies.append(pltpu.async_copy(t_hbm.at[1], t1_v, sem))
    for c in copies:
        c.wait()

    ii = lax.iota(jnp.int32, 16)
    i2 = ii * 2
    for g in range(_BPW // _L):
        sl = pl.ds(g * _L, _L)
        a0 = jnp.zeros((_L,), jnp.float32)
        a1 = jnp.zeros((_L,), jnp.float32)
        for f in range(_NF):
            iv = idx_v[f, sl] + _ROW128[f]
            a0 = a0 + plsc.load_gather(t0_v, [iv])
            a1 = a1 + plsc.load_gather(t1_v, [iv])
        s20 = (ii + g * _L) * _HIST
        for h in range(_HIST):
            ui = plsc.load_gather(u_v, [s20 + h])
            tidx = ui + _UROW128
            a0 = a0 + plsc.load_gather(t0_v, [tidx])
            a1 = a1 + plsc.load_gather(t1_v, [tidx])
        m = jnp.maximum(a0, a1)
        e = jnp.exp(-jnp.abs(a0 - a1))
        z = e / (e + 2.0)
        z2 = z * z
        lse = m + 2.0 * z * (1.0 + z2 * (
            (1.0 / 3.0) + z2 * (0.2 + z2 * ((1.0 / 7.0) + z2 * (1.0 / 9.0)))))
        plsc.store_scatter(o_v, [i2 + (g * 2 * _L)], a0 - lse)
        plsc.store_scatter(o_v, [i2 + (g * 2 * _L + 1)], a1 - lse)
    pltpu.sync_copy(o_v, out_hbm.at[pl.ds(w * (2 * _BPW), 2 * _BPW)])


@functools.cache
def _make_sc_bag():
    return pl.kernel(
        _sc_bag_body,
        mesh=plsc.VectorSubcoreMesh(core_axis_name="c", subcore_axis_name="s"),
        out_type=jax.ShapeDtypeStruct((2 * _B,), jnp.float32),
        scratch_types=[
            pltpu.VMEM((_NF, _BPW), jnp.int32),
            pltpu.VMEM((_BPW * _HIST,), jnp.int32),
            pltpu.VMEM((_RP2,), jnp.float32),
            pltpu.VMEM((_RP2,), jnp.float32),
            pltpu.VMEM((2 * _BPW,), jnp.float32),
            pltpu.SemaphoreType.DMA,
        ],
        compiler_params=pltpu.CompilerParams(needs_layout_passes=False),
    )


def kernel(ip1_idx, ip1_table, ip2_idx, ip2_table, ip3_idx, ip3_table,
           url_idx, url_table, aurl_idx, aurl_table,
           regionid_idx, regionid_table, cityid_idx, cityid_table,
           adexchange_idx, adexchange_table, adslotw_idx, adslotw_table,
           adsloth_idx, adsloth_table, adslotv_idx, adslotv_table,
           adslotfp_idx, adslotfp_table, creativeid_idx, creativeid_table,
           bidprice_idx, bidprice_table, payprice_idx, payprice_table,
           userids_idx, userids_table, W, b):
    tables = [ip1_table, ip2_table, ip3_table, url_table, aurl_table,
              regionid_table, cityid_table, adexchange_table, adslotw_table,
              adsloth_table, adslotv_table, adslotfp_table, creativeid_table,
              bidprice_table, payprice_table]
    idxs = [ip1_idx, ip2_idx, ip3_idx, url_idx, aurl_idx, regionid_idx,
            cityid_idx, adexchange_idx, adslotw_idx, adsloth_idx, adslotv_idx,
            adslotfp_idx, creativeid_idx, bidprice_idx, payprice_idx]

    t_full = _fuse_tables(W, b.reshape(1, 2), *tables, userids_table)
    idxs32 = [i.astype(jnp.int32) for i in idxs]
    u_flat = userids_idx.astype(jnp.int32).reshape(_B * _HIST)
    out = _make_sc_bag()(*idxs32, u_flat, t_full)
    return out.reshape(_B, 2)
